# baseline (speedup 1.0000x reference)
import functools

import jax
import jax.numpy as jnp
from jax import lax
from jax.experimental import pallas as pl
from jax.experimental.pallas import tpu as pltpu
from jax.experimental.pallas import tpu_sc as plsc

_B, _N, _T = 512, 128, 5
_NG = _N * _T
_NC, _NS, _L = 2, 16, 16
_NW = _NC * _NS
_BPW = _B // _NW
_NCH = _N // _L
_ROW = 2 * _N
_WROW = _BPW * _ROW


def _sc_loss(pred2, gt2):
    mesh = plsc.VectorSubcoreMesh(core_axis_name="c", subcore_axis_name="s")

    @functools.partial(
        pl.kernel,
        mesh=mesh,
        out_type=jax.ShapeDtypeStruct((_NC, _L), jnp.float32),
        compiler_params=pltpu.CompilerParams(needs_layout_passes=False),
        scratch_types=[
            pltpu.VMEM((_WROW,), jnp.float32),
            pltpu.VMEM((_WROW,), jnp.float32),
            pltpu.VMEM((_NG,), jnp.float32),
            pltpu.VMEM((_NG,), jnp.float32),
            pltpu.VMEM((_L,), jnp.float32),
            pltpu.VMEM((_NS * _L,), jnp.float32),
            pltpu.VMEM_SHARED((_NS * _L,), jnp.float32),
        ],
    )
    def k(pred_hbm, gt_hbm, out_hbm, pred_v, gt_v, ix_v, iy_v, st_v, red_v,
          shared):
        cid = lax.axis_index("c")
        sid = lax.axis_index("s")
        wid = cid * _NS + sid
        pltpu.sync_copy(pred_hbm.at[wid], pred_v)
        pltpu.sync_copy(gt_hbm.at[wid], gt_v)
        iota = lax.broadcasted_iota(jnp.int32, (_L,), 0)

        def batch_body(i, acc):
            base = jnp.full((_L,), i * _ROW, jnp.int32)
            for c in range(_NCH):
                s_b = iota + (c * _L)
                s_a = (s_b + (_N - 1)) & (_N - 1)
                gbx = plsc.load_gather(gt_v, [base + s_b * 2])
                gby = plsc.load_gather(gt_v, [base + s_b * 2 + 1])
                gax = plsc.load_gather(gt_v, [base + s_a * 2])
                gay = plsc.load_gather(gt_v, [base + s_a * 2 + 1])
                for t in range(_T):
                    st = t / _T
                    if t == 0:
                        vx, vy = gax, gay
                    else:
                        vx = gbx * st + gax * (1.0 - st)
                        vy = gby * st + gay * (1.0 - st)
                    ix_v[pl.ds(t * _N + c * _L, _L)] = vx
                    iy_v[pl.ds(t * _N + c * _L, _L)] = vy
            pxs, pys = [], []
            for c in range(_NCH):
                colx = base + iota * 2 + (c * 2 * _L)
                pxs.append(plsc.load_gather(pred_v, [colx]))
                pys.append(plsc.load_gather(pred_v, [colx + 1]))

            big = jnp.full((_L,), 1e30, jnp.float32)
            zero_i = jnp.zeros((_L,), jnp.int32)
            carry0 = tuple([big] * _NCH) + tuple([zero_i] * _NCH)

            def cand_body(cc, carry):
                minds = list(carry[:_NCH])
                bidxs = list(carry[_NCH:])
                jbase = cc * _L
                cx = ix_v[pl.ds(jbase, _L)]
                cy = iy_v[pl.ds(jbase, _L)]
                for lane in range(_L):
                    vix = jnp.full((_L,), cx[lane])
                    viy = jnp.full((_L,), cy[lane])
                    vj = jnp.full((_L,), jbase + lane, jnp.int32)
                    for c in range(_NCH):
                        dx = pxs[c] - vix
                        dy = pys[c] - viy
                        d = dx * dx + dy * dy
                        m = d < minds[c]
                        minds[c] = jnp.where(m, d, minds[c])
                        bidxs[c] = jnp.where(m, vj, bidxs[c])
                return tuple(minds) + tuple(bidxs)

            carry = lax.fori_loop(0, _NG // _L, cand_body, carry0)
            bidxs = carry[_NCH:]
            for c in range(_NCH):
                bx = plsc.load_gather(ix_v, [bidxs[c]])
                by = plsc.load_gather(iy_v, [bidxs[c]])
                dx = pxs[c] - bx
                dy = pys[c] - by
                adx = jnp.abs(dx)
                ady = jnp.abs(dy)
                lx = jnp.where(adx < 1.0, 0.5 * dx * dx, adx - 0.5)
                ly = jnp.where(ady < 1.0, 0.5 * dy * dy, ady - 0.5)
                acc = acc + lx + ly
            return acc

        acc = lax.fori_loop(0, _BPW, batch_body,
                            jnp.zeros((_L,), jnp.float32))
        st_v[...] = acc
        pltpu.sync_copy(st_v, shared.at[pl.ds(sid * _L, _L)])
        plsc.subcore_barrier()

        @pl.when(sid == 0)
        def _():
            pltpu.sync_copy(shared, red_v)
            tot = jnp.zeros((_L,), jnp.float32)
            for s in range(_NS):
                tot = tot + red_v[pl.ds(s * _L, _L)]
            total = jnp.sum(tot) * (1.0 / (_B * _N * 2))
            st_v[...] = jnp.full((_L,), total)
            pltpu.sync_copy(st_v, out_hbm.at[cid])

    return k(pred2, gt2)


def kernel(init_polys, pred_poly, gt_polys):
    del init_polys
    pred2 = pred_poly.reshape(_NW, _WROW)
    gt2 = gt_polys.reshape(_NW, _WROW)
    out = _sc_loss(pred2, gt2)
    return out[0, 0] + out[1, 0]
